# Initial kernel scaffold; baseline (speedup 1.0000x reference)
import jax
import jax.numpy as jnp
from jax.experimental import pallas as pl


def kernel(x, bn0_g, bn0_b, conv0_w, conv0_b, blk0_db_bn_g, blk0_db_bn_b, blk0_db_conv_w, blk0_tr_bn_g, blk0_tr_bn_b, blk0_tr_conv_w, blk1_db_bn_g, blk1_db_bn_b, blk1_db_conv_w, blk1_tr_bn_g, blk1_tr_bn_b, blk1_tr_conv_w, blk2_db_bn_g, blk2_db_bn_b, blk2_db_conv_w, blk2_tr_bn_g, blk2_tr_bn_b, blk2_tr_conv_w, blk3_db_bn_g, blk3_db_bn_b, blk3_db_conv_w, blk3_tr_bn_g, blk3_tr_bn_b, blk3_tr_conv_w, mixI_w, mixI_b, mixW_w, mixW_b):
    raise NotImplementedError("write your pallas kernel here")


# speedup vs baseline: 2.1551x; 2.1551x over previous
import functools

import jax
import jax.numpy as jnp
from jax import lax
from jax.experimental import pallas as pl
from jax.experimental.pallas import tpu as pltpu

_BN_EPS = 1e-5
_SLOPE = 0.2
_VMEM_LIMIT = 64 * 1024 * 1024
_CFG = ((6, 2), (12, 2), (24, 2), (16, 4))


def _call(kernel_fn, ins, out_shapes):
    def spec(a):
        nd = len(a.shape)
        return pl.BlockSpec(a.shape, lambda i, _nd=nd: (0,) * _nd)
    single = not isinstance(out_shapes, (tuple, list))
    outs = [out_shapes] if single else list(out_shapes)
    res = pl.pallas_call(
        kernel_fn,
        out_shape=outs,
        grid_spec=pltpu.PrefetchScalarGridSpec(
            num_scalar_prefetch=0,
            grid=(1,),
            in_specs=[spec(a) for a in ins],
            out_specs=[spec(o) for o in outs],
        ),
        compiler_params=pltpu.CompilerParams(
            dimension_semantics=("arbitrary",),
            vmem_limit_bytes=_VMEM_LIMIT,
        ),
    )(*ins)
    return res[0] if single else res


def _roll_l(x, k):
    n = x.shape[-1]
    k %= n
    if k == 0:
        return x
    return jnp.concatenate([x[:, k:], x[:, :k]], axis=1)


def _bn(x, g, b):
    mean = jnp.mean(x, axis=1, keepdims=True)
    cent = x - mean
    var = jnp.mean(cent * cent, axis=1, keepdims=True)
    return cent * (lax.rsqrt(var + _BN_EPS) * g) + b


def _conv3x3(xa, w_ref, cin, h, w):
    n = xa.shape[-1]
    p = lax.broadcasted_iota(jnp.int32, (1, n), 1)
    jj = p % w
    ii = (p // w) % h
    acc = None
    for di in range(3):
        for dj in range(3):
            t = 3 * di + dj
            shifted = _roll_l(xa, (di - 1) * w + (dj - 1))
            conds = []
            if di == 0:
                conds.append(ii >= 1)
            elif di == 2:
                conds.append(ii < h - 1)
            if dj == 0:
                conds.append(jj >= 1)
            elif dj == 2:
                conds.append(jj < w - 1)
            if conds:
                m = conds[0]
                for c in conds[1:]:
                    m = jnp.logical_and(m, c)
                shifted = jnp.where(m, shifted, 0.0)
            d = jnp.dot(w_ref[:, t * cin:(t + 1) * cin], shifted,
                        preferred_element_type=jnp.float32)
            acc = d if acc is None else acc + d
    return acc


def _pool_max(t, w, mp):
    m = t
    sh = 1
    while sh < mp:
        m = jnp.maximum(m, _roll_l(m, sh))
        sh *= 2
    sh = w
    while sh < mp * w:
        m = jnp.maximum(m, _roll_l(m, sh))
        sh *= 2
    return m


def _block_body(x, g1, b1, w3_ref, g2, b2, w1, h, w, mp):
    cin = x.shape[0]
    xa = jnp.maximum(_bn(x, g1, b1), 0.0)
    acc = _conv3x3(xa, w3_ref, cin, h, w)
    z = _bn(acc, g2, b2)
    z = jnp.where(z >= 0.0, z, _SLOPE * z)
    t = jnp.dot(w1, z, preferred_element_type=jnp.float32)
    return _pool_max(t, w, mp)


def _stem_bn_kernel(x_ref, g_ref, b_ref, o_ref):
    o_ref[...] = jnp.maximum(_bn(x_ref[...], g_ref[...], b_ref[...]), 0.0)


def _stem_block0_kernel(p_ref, sw_ref, sb_ref, g1_ref, b1_ref, w3_ref,
                        g2_ref, b2_ref, w1_ref, o_ref, *, h, w, mp):
    x0 = jnp.dot(sw_ref[...], p_ref[...],
                 preferred_element_type=jnp.float32) + sb_ref[...]
    o_ref[...] = _block_body(x0, g1_ref[...], b1_ref[...], w3_ref,
                             g2_ref[...], b2_ref[...], w1_ref[...], h, w, mp)


def _block_kernel(x_ref, g1_ref, b1_ref, w3_ref, g2_ref, b2_ref, w1_ref,
                  o_ref, *, h, w, mp):
    o_ref[...] = _block_body(x_ref[...], g1_ref[...], b1_ref[...], w3_ref,
                             g2_ref[...], b2_ref[...], w1_ref[...], h, w, mp)


def _final_kernel(x_ref, g1_ref, b1_ref, w3_ref, g2_ref, b2_ref, w1_ref,
                  wm_ref, bm_ref, z_ref, o_ref, *, h, w, mp, batch):
    m = _block_body(x_ref[...], g1_ref[...], b1_ref[...], w3_ref,
                    g2_ref[...], b2_ref[...], w1_ref[...], h, w, mp)
    n = m.shape[-1]
    pi = lax.broadcasted_iota(jnp.int32, (n, batch), 0)
    bi = lax.broadcasted_iota(jnp.int32, (n, batch), 1)
    sel = (pi == bi * (h * w)).astype(jnp.float32)
    zmat = jax.nn.sigmoid(jnp.dot(m, sel, preferred_element_type=jnp.float32))
    z_ref[...] = zmat
    o_ref[...] = jax.nn.sigmoid(
        jnp.dot(wm_ref[...], zmat, preferred_element_type=jnp.float32)
        + bm_ref[...])


def _im2col(x_cbhw, ksize, stride, padding):
    c, b, h, w = x_cbhw.shape
    ho = (h + 2 * padding - ksize) // stride + 1
    wo = (w + 2 * padding - ksize) // stride + 1
    if padding > 0:
        x_cbhw = jnp.pad(
            x_cbhw, ((0, 0), (0, 0), (padding, padding), (padding, padding)))
    cols = []
    for ki in range(ksize):
        for kj in range(ksize):
            cols.append(
                x_cbhw[:, :, ki:ki + stride * ho:stride, kj:kj + stride * wo:stride])
    return jnp.stack(cols, axis=1).reshape(c * ksize * ksize, b * ho * wo), ho, wo


def _subsample(flat, c, b, h, w, mp):
    x4 = flat.reshape(c, b, h, w)[:, :, ::mp, ::mp]
    return x4.reshape(c, b * (h // mp) * (w // mp))


def _tap_major(w_conv, scale):
    c_out, c_in = w_conv.shape[0], w_conv.shape[1]
    wt = jnp.transpose(w_conv, (0, 2, 3, 1)).reshape(c_out, 9 * c_in)
    return (wt * scale).astype(jnp.float32)


def kernel(x, bn0_g, bn0_b, conv0_w, conv0_b, blk0_db_bn_g, blk0_db_bn_b, blk0_db_conv_w, blk0_tr_bn_g, blk0_tr_bn_b, blk0_tr_conv_w, blk1_db_bn_g, blk1_db_bn_b, blk1_db_conv_w, blk1_tr_bn_g, blk1_tr_bn_b, blk1_tr_conv_w, blk2_db_bn_g, blk2_db_bn_b, blk2_db_conv_w, blk2_tr_bn_g, blk2_tr_bn_b, blk2_tr_conv_w, blk3_db_bn_g, blk3_db_bn_b, blk3_db_conv_w, blk3_tr_bn_g, blk3_tr_bn_b, blk3_tr_conv_w, mixI_w, mixI_b, mixW_w, mixW_b):
    f32 = jnp.float32
    batch, nc, hin, win = x.shape
    blocks = (
        (blk0_db_bn_g, blk0_db_bn_b, blk0_db_conv_w, blk0_tr_bn_g, blk0_tr_bn_b, blk0_tr_conv_w),
        (blk1_db_bn_g, blk1_db_bn_b, blk1_db_conv_w, blk1_tr_bn_g, blk1_tr_bn_b, blk1_tr_conv_w),
        (blk2_db_bn_g, blk2_db_bn_b, blk2_db_conv_w, blk2_tr_bn_g, blk2_tr_bn_b, blk2_tr_conv_w),
        (blk3_db_bn_g, blk3_db_bn_b, blk3_db_conv_w, blk3_tr_bn_g, blk3_tr_bn_b, blk3_tr_conv_w),
    )

    xc = jnp.transpose(x, (1, 0, 2, 3)).astype(f32).reshape(nc, batch * hin * win)
    xbn = _call(_stem_bn_kernel,
                (xc, bn0_g.reshape(nc, 1).astype(f32), bn0_b.reshape(nc, 1).astype(f32)),
                jax.ShapeDtypeStruct(xc.shape, f32))

    patches, h, w = _im2col(xbn.reshape(nc, batch, hin, win), 4, 2, 1)
    c_out0 = conv0_w.shape[0]
    sw = conv0_w.reshape(c_out0, nc * 16).astype(f32)
    sb = conv0_b.reshape(c_out0, 1).astype(f32)

    def block_params(i):
        g1, b1, w3, g2, b2, w1 = blocks[i]
        n_convs, mp = _CFG[i]
        scale = 2.0 ** (n_convs - 2) if n_convs >= 2 else 1.0
        c_in = w3.shape[0]
        c_nxt = w1.shape[0]
        return (g1.reshape(c_in, 1).astype(f32), b1.reshape(c_in, 1).astype(f32),
                _tap_major(w3, scale),
                g2.reshape(c_in, 1).astype(f32), b2.reshape(c_in, 1).astype(f32),
                w1.reshape(c_nxt, c_in).astype(f32), mp, c_in, c_nxt)

    g1, b1, w3, g2, b2, w1, mp, c_in, c_nxt = block_params(0)
    kfn = functools.partial(_stem_block0_kernel, h=h, w=w, mp=mp)
    full = _call(kfn, (patches, sw, sb, g1, b1, w3, g2, b2, w1),
                 jax.ShapeDtypeStruct((c_nxt, batch * h * w), f32))
    cur = _subsample(full, c_nxt, batch, h, w, mp)
    h, w = h // mp, w // mp

    for i in (1, 2):
        g1, b1, w3, g2, b2, w1, mp, c_in, c_nxt = block_params(i)
        kfn = functools.partial(_block_kernel, h=h, w=w, mp=mp)
        full = _call(kfn, (cur, g1, b1, w3, g2, b2, w1),
                     jax.ShapeDtypeStruct((c_nxt, batch * h * w), f32))
        cur = _subsample(full, c_nxt, batch, h, w, mp)
        h, w = h // mp, w // mp

    g1, b1, w3, g2, b2, w1, mp, c_in, zdim = block_params(3)
    idim = mixI_w.shape[0]
    wdim = mixW_w.shape[0]
    wm = jnp.concatenate([mixI_w, mixW_w], axis=0).astype(f32)
    bm = jnp.concatenate([mixI_b, mixW_b], axis=0).reshape(idim + wdim, 1).astype(f32)
    kfn = functools.partial(_final_kernel, h=h, w=w, mp=mp, batch=batch)
    zmat, out_t = _call(kfn, (cur, g1, b1, w3, g2, b2, w1, wm, bm),
                        [jax.ShapeDtypeStruct((zdim, batch), f32),
                         jax.ShapeDtypeStruct((idim + wdim, batch), f32)])

    z = jnp.transpose(zmat)
    z_img = jnp.transpose(out_t[:idim])
    z_warp = jnp.transpose(out_t[idim:])
    return z, z_img, z_warp


# speedup vs baseline: 2.2692x; 1.0529x over previous
import functools

import jax
import jax.numpy as jnp
from jax import lax
from jax.experimental import pallas as pl
from jax.experimental.pallas import tpu as pltpu

_BN_EPS = 1e-5
_SLOPE = 0.2
_VMEM_LIMIT = 64 * 1024 * 1024
_CFG = ((6, 2), (12, 2), (24, 2), (16, 4))


def _call(kernel_fn, ins, out_shapes):
    def spec(a):
        nd = len(a.shape)
        return pl.BlockSpec(a.shape, lambda i, _nd=nd: (0,) * _nd)
    single = not isinstance(out_shapes, (tuple, list))
    outs = [out_shapes] if single else list(out_shapes)
    res = pl.pallas_call(
        kernel_fn,
        out_shape=outs,
        grid_spec=pltpu.PrefetchScalarGridSpec(
            num_scalar_prefetch=0,
            grid=(1,),
            in_specs=[spec(a) for a in ins],
            out_specs=[spec(o) for o in outs],
        ),
        compiler_params=pltpu.CompilerParams(
            dimension_semantics=("arbitrary",),
            vmem_limit_bytes=_VMEM_LIMIT,
        ),
    )(*ins)
    return res[0] if single else res


def _roll_l(x, k):
    n = x.shape[-1]
    k %= n
    if k == 0:
        return x
    return jnp.concatenate([x[:, k:], x[:, :k]], axis=1)


def _bn(x, g, b):
    mean = jnp.mean(x, axis=1, keepdims=True)
    cent = x - mean
    var = jnp.mean(cent * cent, axis=1, keepdims=True)
    return cent * (lax.rsqrt(var + _BN_EPS) * g) + b


def _conv3x3(xa, w_ref, cin, h, w):
    n = xa.shape[-1]
    p = lax.broadcasted_iota(jnp.int32, (1, n), 1)
    jj = p % w
    ii = (p // w) % h
    xa = xa.astype(jnp.bfloat16)
    acc = None
    for di in range(3):
        for dj in range(3):
            t = 3 * di + dj
            shifted = _roll_l(xa, (di - 1) * w + (dj - 1))
            conds = []
            if di == 0:
                conds.append(ii >= 1)
            elif di == 2:
                conds.append(ii < h - 1)
            if dj == 0:
                conds.append(jj >= 1)
            elif dj == 2:
                conds.append(jj < w - 1)
            if conds:
                m = conds[0]
                for c in conds[1:]:
                    m = jnp.logical_and(m, c)
                shifted = jnp.where(m, shifted, jnp.bfloat16(0.0))
            d = jnp.dot(w_ref[:, t * cin:(t + 1) * cin], shifted,
                        preferred_element_type=jnp.float32)
            acc = d if acc is None else acc + d
    return acc


def _pool_max(t, w, mp):
    m = t
    sh = 1
    while sh < mp:
        m = jnp.maximum(m, _roll_l(m, sh))
        sh *= 2
    sh = w
    while sh < mp * w:
        m = jnp.maximum(m, _roll_l(m, sh))
        sh *= 2
    return m


def _block_body(x, g1, b1, w3_ref, g2, b2, w1, h, w, mp):
    cin = x.shape[0]
    xa = jnp.maximum(_bn(x, g1, b1), 0.0)
    acc = _conv3x3(xa, w3_ref, cin, h, w)
    z = _bn(acc, g2, b2)
    z = jnp.where(z >= 0.0, z, _SLOPE * z)
    t = jnp.dot(w1, z.astype(jnp.bfloat16), preferred_element_type=jnp.float32)
    return _pool_max(t, w, mp)


def _stem_bn_kernel(x_ref, g_ref, b_ref, o_ref):
    o_ref[...] = jnp.maximum(_bn(x_ref[...], g_ref[...], b_ref[...]), 0.0)


def _stem_block0_kernel(p_ref, sw_ref, sb_ref, g1_ref, b1_ref, w3_ref,
                        g2_ref, b2_ref, w1_ref, o_ref, *, h, w, mp):
    x0 = jnp.dot(sw_ref[...], p_ref[...].astype(jnp.bfloat16),
                 preferred_element_type=jnp.float32) + sb_ref[...]
    o_ref[...] = _block_body(x0, g1_ref[...], b1_ref[...], w3_ref,
                             g2_ref[...], b2_ref[...], w1_ref[...], h, w, mp)


def _block_kernel(x_ref, g1_ref, b1_ref, w3_ref, g2_ref, b2_ref, w1_ref,
                  o_ref, *, h, w, mp):
    o_ref[...] = _block_body(x_ref[...], g1_ref[...], b1_ref[...], w3_ref,
                             g2_ref[...], b2_ref[...], w1_ref[...], h, w, mp)


def _final_kernel(x_ref, g1_ref, b1_ref, w3_ref, g2_ref, b2_ref, w1_ref,
                  wm_ref, bm_ref, z_ref, o_ref, *, h, w, mp, batch):
    m = _block_body(x_ref[...], g1_ref[...], b1_ref[...], w3_ref,
                    g2_ref[...], b2_ref[...], w1_ref[...], h, w, mp)
    n = m.shape[-1]
    pi = lax.broadcasted_iota(jnp.int32, (n, batch), 0)
    bi = lax.broadcasted_iota(jnp.int32, (n, batch), 1)
    sel = (pi == bi * (h * w)).astype(jnp.float32)
    zmat = jax.nn.sigmoid(jnp.dot(m, sel, preferred_element_type=jnp.float32))
    z_ref[...] = zmat
    o_ref[...] = jax.nn.sigmoid(
        jnp.dot(wm_ref[...], zmat, preferred_element_type=jnp.float32)
        + bm_ref[...])


def _im2col(x_cbhw, ksize, stride, padding):
    c, b, h, w = x_cbhw.shape
    ho = (h + 2 * padding - ksize) // stride + 1
    wo = (w + 2 * padding - ksize) // stride + 1
    if padding > 0:
        x_cbhw = jnp.pad(
            x_cbhw, ((0, 0), (0, 0), (padding, padding), (padding, padding)))
    cols = []
    for ki in range(ksize):
        for kj in range(ksize):
            cols.append(
                x_cbhw[:, :, ki:ki + stride * ho:stride, kj:kj + stride * wo:stride])
    return jnp.stack(cols, axis=1).reshape(c * ksize * ksize, b * ho * wo), ho, wo


def _subsample(flat, c, b, h, w, mp):
    x4 = flat.reshape(c, b, h, w)[:, :, ::mp, ::mp]
    return x4.reshape(c, b * (h // mp) * (w // mp))


def _tap_major(w_conv, scale):
    c_out, c_in = w_conv.shape[0], w_conv.shape[1]
    wt = jnp.transpose(w_conv, (0, 2, 3, 1)).reshape(c_out, 9 * c_in)
    return (wt * scale).astype(jnp.bfloat16)


def kernel(x, bn0_g, bn0_b, conv0_w, conv0_b, blk0_db_bn_g, blk0_db_bn_b, blk0_db_conv_w, blk0_tr_bn_g, blk0_tr_bn_b, blk0_tr_conv_w, blk1_db_bn_g, blk1_db_bn_b, blk1_db_conv_w, blk1_tr_bn_g, blk1_tr_bn_b, blk1_tr_conv_w, blk2_db_bn_g, blk2_db_bn_b, blk2_db_conv_w, blk2_tr_bn_g, blk2_tr_bn_b, blk2_tr_conv_w, blk3_db_bn_g, blk3_db_bn_b, blk3_db_conv_w, blk3_tr_bn_g, blk3_tr_bn_b, blk3_tr_conv_w, mixI_w, mixI_b, mixW_w, mixW_b):
    f32 = jnp.float32
    batch, nc, hin, win = x.shape
    blocks = (
        (blk0_db_bn_g, blk0_db_bn_b, blk0_db_conv_w, blk0_tr_bn_g, blk0_tr_bn_b, blk0_tr_conv_w),
        (blk1_db_bn_g, blk1_db_bn_b, blk1_db_conv_w, blk1_tr_bn_g, blk1_tr_bn_b, blk1_tr_conv_w),
        (blk2_db_bn_g, blk2_db_bn_b, blk2_db_conv_w, blk2_tr_bn_g, blk2_tr_bn_b, blk2_tr_conv_w),
        (blk3_db_bn_g, blk3_db_bn_b, blk3_db_conv_w, blk3_tr_bn_g, blk3_tr_bn_b, blk3_tr_conv_w),
    )

    xc = jnp.transpose(x, (1, 0, 2, 3)).astype(f32).reshape(nc, batch * hin * win)
    xbn = _call(_stem_bn_kernel,
                (xc, bn0_g.reshape(nc, 1).astype(f32), bn0_b.reshape(nc, 1).astype(f32)),
                jax.ShapeDtypeStruct(xc.shape, f32))

    patches, h, w = _im2col(xbn.reshape(nc, batch, hin, win), 4, 2, 1)
    c_out0 = conv0_w.shape[0]
    sw = conv0_w.reshape(c_out0, nc * 16).astype(jnp.bfloat16)
    sb = conv0_b.reshape(c_out0, 1).astype(f32)

    def block_params(i):
        g1, b1, w3, g2, b2, w1 = blocks[i]
        n_convs, mp = _CFG[i]
        scale = 2.0 ** (n_convs - 2) if n_convs >= 2 else 1.0
        c_in = w3.shape[0]
        c_nxt = w1.shape[0]
        return (g1.reshape(c_in, 1).astype(f32), b1.reshape(c_in, 1).astype(f32),
                _tap_major(w3, scale),
                g2.reshape(c_in, 1).astype(f32), b2.reshape(c_in, 1).astype(f32),
                w1.reshape(c_nxt, c_in).astype(jnp.bfloat16), mp, c_in, c_nxt)

    g1, b1, w3, g2, b2, w1, mp, c_in, c_nxt = block_params(0)
    kfn = functools.partial(_stem_block0_kernel, h=h, w=w, mp=mp)
    full = _call(kfn, (patches, sw, sb, g1, b1, w3, g2, b2, w1),
                 jax.ShapeDtypeStruct((c_nxt, batch * h * w), f32))
    cur = _subsample(full, c_nxt, batch, h, w, mp)
    h, w = h // mp, w // mp

    for i in (1, 2):
        g1, b1, w3, g2, b2, w1, mp, c_in, c_nxt = block_params(i)
        kfn = functools.partial(_block_kernel, h=h, w=w, mp=mp)
        full = _call(kfn, (cur, g1, b1, w3, g2, b2, w1),
                     jax.ShapeDtypeStruct((c_nxt, batch * h * w), f32))
        cur = _subsample(full, c_nxt, batch, h, w, mp)
        h, w = h // mp, w // mp

    g1, b1, w3, g2, b2, w1, mp, c_in, zdim = block_params(3)
    idim = mixI_w.shape[0]
    wdim = mixW_w.shape[0]
    wm = jnp.concatenate([mixI_w, mixW_w], axis=0).astype(f32)
    bm = jnp.concatenate([mixI_b, mixW_b], axis=0).reshape(idim + wdim, 1).astype(f32)
    kfn = functools.partial(_final_kernel, h=h, w=w, mp=mp, batch=batch)
    zmat, out_t = _call(kfn, (cur, g1, b1, w3, g2, b2, w1, wm, bm),
                        [jax.ShapeDtypeStruct((zdim, batch), f32),
                         jax.ShapeDtypeStruct((idim + wdim, batch), f32)])

    z = jnp.transpose(zmat)
    z_img = jnp.transpose(out_t[:idim])
    z_warp = jnp.transpose(out_t[idim:])
    return z, z_img, z_warp


# speedup vs baseline: 3.1771x; 1.4001x over previous
import functools

import jax
import jax.numpy as jnp
from jax import lax
from jax.experimental import pallas as pl
from jax.experimental.pallas import tpu as pltpu

_BN_EPS = 1e-5
_SLOPE = 0.2
_VMEM_LIMIT = 64 * 1024 * 1024
_CFG = ((6, 2), (12, 2), (24, 2), (16, 4))


def _call(kernel_fn, ins, out_shapes):
    def spec(a):
        nd = len(a.shape)
        return pl.BlockSpec(a.shape, lambda i, _nd=nd: (0,) * _nd)
    single = not isinstance(out_shapes, (tuple, list))
    outs = [out_shapes] if single else list(out_shapes)
    res = pl.pallas_call(
        kernel_fn,
        out_shape=outs,
        grid_spec=pltpu.PrefetchScalarGridSpec(
            num_scalar_prefetch=0,
            grid=(1,),
            in_specs=[spec(a) for a in ins],
            out_specs=[spec(o) for o in outs],
        ),
        compiler_params=pltpu.CompilerParams(
            dimension_semantics=("arbitrary",),
            vmem_limit_bytes=_VMEM_LIMIT,
        ),
    )(*ins)
    return res[0] if single else res


def _roll_l(x, k):
    n = x.shape[-1]
    k %= n
    if k == 0:
        return x
    return jnp.concatenate([x[:, k:], x[:, :k]], axis=1)


def _bn(x, g, b):
    xf = x.astype(jnp.float32)
    mean = jnp.mean(xf, axis=1, keepdims=True)
    ex2 = jnp.mean(xf * xf, axis=1, keepdims=True)
    var = ex2 - mean * mean
    a = lax.rsqrt(var + _BN_EPS) * g
    return xf * a + (b - mean * a)


def _conv3x3(xa, w_ref, cin, h, w):
    n = xa.shape[-1]
    p = lax.broadcasted_iota(jnp.int32, (1, n), 1)
    jj = p % w
    ii = (p // w) % h
    xa = xa.astype(jnp.bfloat16)
    acc = None
    for di in range(3):
        for dj in range(3):
            t = 3 * di + dj
            shifted = _roll_l(xa, (di - 1) * w + (dj - 1))
            conds = []
            if di == 0:
                conds.append(ii >= 1)
            elif di == 2:
                conds.append(ii < h - 1)
            if dj == 0:
                conds.append(jj >= 1)
            elif dj == 2:
                conds.append(jj < w - 1)
            if conds:
                m = conds[0]
                for c in conds[1:]:
                    m = jnp.logical_and(m, c)
                shifted = jnp.where(m, shifted, jnp.bfloat16(0.0))
            d = jnp.dot(w_ref[:, t * cin:(t + 1) * cin], shifted,
                        preferred_element_type=jnp.float32)
            acc = d if acc is None else acc + d
    return acc


def _pool_max(t, w, mp):
    m = t
    sh = 1
    while sh < mp:
        m = jnp.maximum(m, _roll_l(m, sh))
        sh *= 2
    sh = w
    while sh < mp * w:
        m = jnp.maximum(m, _roll_l(m, sh))
        sh *= 2
    return m


def _block_body(x, g1, b1, w3_ref, g2, b2, w1, h, w, mp, pool_dtype):
    cin = x.shape[0]
    xa = jnp.maximum(_bn(x, g1, b1), 0.0)
    acc = _conv3x3(xa, w3_ref, cin, h, w)
    z = _bn(acc, g2, b2)
    z = jnp.where(z >= 0.0, z, _SLOPE * z).astype(jnp.bfloat16)
    t = jnp.dot(w1, z, preferred_element_type=jnp.float32)
    return _pool_max(t.astype(pool_dtype), w, mp)


def _stem_bn_kernel(x_ref, g_ref, b_ref, o_ref):
    o_ref[...] = jnp.maximum(
        _bn(x_ref[...], g_ref[...], b_ref[...]), 0.0).astype(jnp.bfloat16)


def _stem_block0_kernel(p_ref, sw_ref, sb_ref, g1_ref, b1_ref, w3_ref,
                        g2_ref, b2_ref, w1_ref, o_ref, *, h, w, mp):
    x0 = jnp.dot(sw_ref[...], p_ref[...],
                 preferred_element_type=jnp.float32) + sb_ref[...]
    o_ref[...] = _block_body(x0, g1_ref[...], b1_ref[...], w3_ref,
                             g2_ref[...], b2_ref[...], w1_ref[...], h, w, mp,
                             jnp.bfloat16)


def _block_kernel(x_ref, g1_ref, b1_ref, w3_ref, g2_ref, b2_ref, w1_ref,
                  o_ref, *, h, w, mp):
    o_ref[...] = _block_body(x_ref[...], g1_ref[...], b1_ref[...], w3_ref,
                             g2_ref[...], b2_ref[...], w1_ref[...], h, w, mp,
                             jnp.bfloat16)


def _final_kernel(x_ref, g1_ref, b1_ref, w3_ref, g2_ref, b2_ref, w1_ref,
                  wm_ref, bm_ref, z_ref, o_ref, *, h, w, mp, batch):
    m = _block_body(x_ref[...], g1_ref[...], b1_ref[...], w3_ref,
                    g2_ref[...], b2_ref[...], w1_ref[...], h, w, mp,
                    jnp.float32)
    n = m.shape[-1]
    pi = lax.broadcasted_iota(jnp.int32, (n, batch), 0)
    bi = lax.broadcasted_iota(jnp.int32, (n, batch), 1)
    sel = (pi == bi * (h * w)).astype(jnp.float32)
    zmat = jax.nn.sigmoid(jnp.dot(m, sel, preferred_element_type=jnp.float32))
    z_ref[...] = zmat
    o_ref[...] = jax.nn.sigmoid(
        jnp.dot(wm_ref[...], zmat, preferred_element_type=jnp.float32)
        + bm_ref[...])


def _im2col(x_cbhw, ksize, stride, padding):
    c, b, h, w = x_cbhw.shape
    ho = (h + 2 * padding - ksize) // stride + 1
    wo = (w + 2 * padding - ksize) // stride + 1
    if padding > 0:
        x_cbhw = jnp.pad(
            x_cbhw, ((0, 0), (0, 0), (padding, padding), (padding, padding)))
    cols = []
    for ki in range(ksize):
        for kj in range(ksize):
            cols.append(
                x_cbhw[:, :, ki:ki + stride * ho:stride, kj:kj + stride * wo:stride])
    return jnp.stack(cols, axis=1).reshape(c * ksize * ksize, b * ho * wo), ho, wo


def _subsample(flat, c, b, h, w, mp):
    x4 = flat.reshape(c, b, h, w)[:, :, ::mp, ::mp]
    return x4.reshape(c, b * (h // mp) * (w // mp))


def _tap_major(w_conv, scale):
    c_out, c_in = w_conv.shape[0], w_conv.shape[1]
    wb = w_conv.astype(jnp.bfloat16) * jnp.bfloat16(scale)
    return jnp.transpose(wb, (0, 2, 3, 1)).reshape(c_out, 9 * c_in)


def kernel(x, bn0_g, bn0_b, conv0_w, conv0_b, blk0_db_bn_g, blk0_db_bn_b, blk0_db_conv_w, blk0_tr_bn_g, blk0_tr_bn_b, blk0_tr_conv_w, blk1_db_bn_g, blk1_db_bn_b, blk1_db_conv_w, blk1_tr_bn_g, blk1_tr_bn_b, blk1_tr_conv_w, blk2_db_bn_g, blk2_db_bn_b, blk2_db_conv_w, blk2_tr_bn_g, blk2_tr_bn_b, blk2_tr_conv_w, blk3_db_bn_g, blk3_db_bn_b, blk3_db_conv_w, blk3_tr_bn_g, blk3_tr_bn_b, blk3_tr_conv_w, mixI_w, mixI_b, mixW_w, mixW_b):
    f32 = jnp.float32
    batch, nc, hin, win = x.shape
    blocks = (
        (blk0_db_bn_g, blk0_db_bn_b, blk0_db_conv_w, blk0_tr_bn_g, blk0_tr_bn_b, blk0_tr_conv_w),
        (blk1_db_bn_g, blk1_db_bn_b, blk1_db_conv_w, blk1_tr_bn_g, blk1_tr_bn_b, blk1_tr_conv_w),
        (blk2_db_bn_g, blk2_db_bn_b, blk2_db_conv_w, blk2_tr_bn_g, blk2_tr_bn_b, blk2_tr_conv_w),
        (blk3_db_bn_g, blk3_db_bn_b, blk3_db_conv_w, blk3_tr_bn_g, blk3_tr_bn_b, blk3_tr_conv_w),
    )

    xc = jnp.transpose(x, (1, 0, 2, 3)).astype(f32).reshape(nc, batch * hin * win)
    xbn = _call(_stem_bn_kernel,
                (xc, bn0_g.reshape(nc, 1).astype(f32), bn0_b.reshape(nc, 1).astype(f32)),
                jax.ShapeDtypeStruct(xc.shape, jnp.bfloat16))

    patches, h, w = _im2col(xbn.reshape(nc, batch, hin, win), 4, 2, 1)
    c_out0 = conv0_w.shape[0]
    sw = conv0_w.reshape(c_out0, nc * 16).astype(jnp.bfloat16)
    sb = conv0_b.reshape(c_out0, 1).astype(f32)

    def block_params(i):
        g1, b1, w3, g2, b2, w1 = blocks[i]
        n_convs, mp = _CFG[i]
        scale = 2.0 ** (n_convs - 2) if n_convs >= 2 else 1.0
        c_in = w3.shape[0]
        c_nxt = w1.shape[0]
        return (g1.reshape(c_in, 1).astype(f32), b1.reshape(c_in, 1).astype(f32),
                _tap_major(w3, scale),
                g2.reshape(c_in, 1).astype(f32), b2.reshape(c_in, 1).astype(f32),
                w1.reshape(c_nxt, c_in).astype(jnp.bfloat16), mp, c_in, c_nxt)

    g1, b1, w3, g2, b2, w1, mp, c_in, c_nxt = block_params(0)
    kfn = functools.partial(_stem_block0_kernel, h=h, w=w, mp=mp)
    full = _call(kfn, (patches, sw, sb, g1, b1, w3, g2, b2, w1),
                 jax.ShapeDtypeStruct((c_nxt, batch * h * w), jnp.bfloat16))
    cur = _subsample(full, c_nxt, batch, h, w, mp)
    h, w = h // mp, w // mp

    for i in (1, 2):
        g1, b1, w3, g2, b2, w1, mp, c_in, c_nxt = block_params(i)
        kfn = functools.partial(_block_kernel, h=h, w=w, mp=mp)
        full = _call(kfn, (cur, g1, b1, w3, g2, b2, w1),
                     jax.ShapeDtypeStruct((c_nxt, batch * h * w), jnp.bfloat16))
        cur = _subsample(full, c_nxt, batch, h, w, mp)
        h, w = h // mp, w // mp

    g1, b1, w3, g2, b2, w1, mp, c_in, zdim = block_params(3)
    idim = mixI_w.shape[0]
    wdim = mixW_w.shape[0]
    wm = jnp.concatenate([mixI_w, mixW_w], axis=0).astype(f32)
    bm = jnp.concatenate([mixI_b, mixW_b], axis=0).reshape(idim + wdim, 1).astype(f32)
    kfn = functools.partial(_final_kernel, h=h, w=w, mp=mp, batch=batch)
    zmat, out_t = _call(kfn, (cur, g1, b1, w3, g2, b2, w1, wm, bm),
                        [jax.ShapeDtypeStruct((zdim, batch), f32),
                         jax.ShapeDtypeStruct((idim + wdim, batch), f32)])

    z = jnp.transpose(zmat)
    z_img = jnp.transpose(out_t[:idim])
    z_warp = jnp.transpose(out_t[idim:])
    return z, z_img, z_warp
